# Initial kernel scaffold; baseline (speedup 1.0000x reference)
import jax
import jax.numpy as jnp
from jax.experimental import pallas as pl


def kernel(nodes, unique_nodes_list, samp_neighs_t, seq_length, features, W_in, b_in):
    raise NotImplementedError("write your pallas kernel here")


# speedup vs baseline: 1.6590x; 1.6590x over previous
import functools

import jax
import jax.numpy as jnp
from jax import lax
from jax.experimental import pallas as pl
from jax.experimental.pallas import tpu as pltpu
from jax.experimental.pallas import tpu_sc as plsc

_NUM_NODES = 10000
_FEAT_DIM = 1433
_K_PAD = 1536
_EMB = 512
_B = 1024
_NUM_SAMPLE = 48
_TOTAL_NEIGHS = _B * _NUM_SAMPLE
_U = 16384

_NC = 2
_NS = 16
_NW = _NC * _NS
_RPW = _TOTAL_NEIGHS // _NW
_NPW = _B // _NW
_CH = 128
_NCH = _RPW // _CH


def _matmul_body(x_ref, w_ref, b_ref, o_ref):
    o_ref[:, :] = (
        jnp.dot(x_ref[:, :], w_ref[:, :], preferred_element_type=jnp.float32)
        + b_ref[:, :]
    )


def _project_features(features_pad, w_pad, b_in):
    bm = 400
    grid = _NUM_NODES // bm
    return pl.pallas_call(
        _matmul_body,
        grid=(grid,),
        in_specs=[
            pl.BlockSpec((bm, _K_PAD), lambda i: (i, 0)),
            pl.BlockSpec((_K_PAD, _EMB), lambda i: (0, 0)),
            pl.BlockSpec((1, _EMB), lambda i: (0, 0)),
        ],
        out_specs=pl.BlockSpec((bm, _EMB), lambda i: (i, 0)),
        out_shape=jax.ShapeDtypeStruct((_NUM_NODES, _EMB), jnp.float32),
    )(features_pad, w_pad, b_in.reshape(1, _EMB))


def _mask_body(seq_ref, m_ref):
    pos = lax.broadcasted_iota(jnp.int32, (_B, _NUM_SAMPLE), 1) + 1
    m_ref[:, :] = pos > seq_ref[:, :]


def _padding_mask(seq_length):
    return pl.pallas_call(
        _mask_body,
        in_specs=[pl.BlockSpec((_B, 1), lambda: (0, 0))],
        out_specs=pl.BlockSpec((_B, _NUM_SAMPLE), lambda: (0, 0)),
        out_shape=jax.ShapeDtypeStruct((_B, _NUM_SAMPLE), jnp.bool_),
    )(seq_length.astype(jnp.int32).reshape(_B, 1))


def _gather_body(
    nodes_hbm,
    unique_hbm,
    samp_hbm,
    proj_hbm,
    nodes_out,
    neighs_out,
    sidx_v,
    cidx_v,
    nidx_v,
    rows_v,
    sem,
):
    wid = lax.axis_index("s") * _NC + lax.axis_index("c")
    base = wid * _RPW

    pltpu.sync_copy(samp_hbm.at[pl.ds(base, _RPW)], sidx_v)

    pltpu.async_copy(unique_hbm.at[sidx_v], cidx_v, sem).wait()

    def chunk(c, carry):
        off = c * _CH
        pltpu.async_copy(
            proj_hbm.at[cidx_v.at[pl.ds(off, _CH)]], rows_v, sem
        ).wait()
        pltpu.sync_copy(rows_v, neighs_out.at[pl.ds(base + off, _CH)])
        return carry

    lax.fori_loop(0, _NCH, chunk, 0)

    nbase = wid * _NPW
    pltpu.sync_copy(nodes_hbm.at[pl.ds(nbase, _NPW)], nidx_v)
    pltpu.async_copy(
        proj_hbm.at[nidx_v], rows_v.at[pl.ds(0, _NPW)], sem
    ).wait()
    pltpu.sync_copy(rows_v.at[pl.ds(0, _NPW)], nodes_out.at[pl.ds(nbase, _NPW)])


def _sc_gather(nodes, unique_nodes_list, samp_flat, proj):
    mesh = plsc.VectorSubcoreMesh(core_axis_name="c", subcore_axis_name="s")
    f = functools.partial(
        pl.kernel,
        mesh=mesh,
        out_type=[
            jax.ShapeDtypeStruct((_B, _EMB), jnp.float32),
            jax.ShapeDtypeStruct((_TOTAL_NEIGHS, _EMB), jnp.float32),
        ],
        scratch_types=[
            pltpu.VMEM((_RPW,), jnp.int32),
            pltpu.VMEM((_RPW,), jnp.int32),
            pltpu.VMEM((_NPW,), jnp.int32),
            pltpu.VMEM((_CH, _EMB), jnp.float32),
            pltpu.SemaphoreType.DMA,
        ],
    )(_gather_body)
    return f(nodes, unique_nodes_list, samp_flat, proj)


def kernel(nodes, unique_nodes_list, samp_neighs_t, seq_length, features, W_in, b_in):
    features_pad = jnp.pad(features, ((0, 0), (0, _K_PAD - _FEAT_DIM)))
    w_pad = jnp.pad(W_in, ((0, _K_PAD - _FEAT_DIM), (0, 0)))
    proj = _project_features(features_pad, w_pad, b_in)

    samp_flat = samp_neighs_t.reshape(-1).astype(jnp.int32)
    nodes_emb, neighs_flat = _sc_gather(
        nodes.astype(jnp.int32),
        unique_nodes_list.astype(jnp.int32),
        samp_flat,
        proj,
    )
    neighs_emb = neighs_flat.reshape(_B, _NUM_SAMPLE, _EMB)
    padding_mask = _padding_mask(seq_length)
    return (nodes_emb, neighs_emb, samp_neighs_t, padding_mask)


# speedup vs baseline: 3.9253x; 2.3661x over previous
import functools

import jax
import jax.numpy as jnp
from jax import lax
from jax.experimental import pallas as pl
from jax.experimental.pallas import tpu as pltpu
from jax.experimental.pallas import tpu_sc as plsc

_NUM_NODES = 10000
_FEAT_DIM = 1433
_K_PAD = 1536
_EMB = 512
_B = 1024
_NUM_SAMPLE = 48
_TOTAL_NEIGHS = _B * _NUM_SAMPLE
_U = 16384

_NC = 2
_NS = 16
_NW = _NC * _NS
_RPW = _TOTAL_NEIGHS // _NW
_NPW = _B // _NW
_CH = 128
_NCH = _RPW // _CH


def _matmul_body(x_ref, w_ref, b_ref, o_ref):
    o_ref[:, :] = (
        jnp.dot(x_ref[:, :], w_ref[:, :], preferred_element_type=jnp.float32)
        + b_ref[:, :]
    )


def _project_features(features, w, b_in):
    bm = 400
    grid = _NUM_NODES // bm
    return pl.pallas_call(
        _matmul_body,
        grid=(grid,),
        in_specs=[
            pl.BlockSpec((bm, _FEAT_DIM), lambda i: (i, 0)),
            pl.BlockSpec((_FEAT_DIM, _EMB), lambda i: (0, 0)),
            pl.BlockSpec((1, _EMB), lambda i: (0, 0)),
        ],
        out_specs=pl.BlockSpec((bm, _EMB), lambda i: (i, 0)),
        out_shape=jax.ShapeDtypeStruct((_NUM_NODES, _EMB), jnp.float32),
    )(features, w, b_in.reshape(1, _EMB))


def _mask_body(seq_ref, m_ref):
    pos = lax.broadcasted_iota(jnp.int32, (_B, _NUM_SAMPLE), 1) + 1
    m_ref[:, :] = pos > seq_ref[:, :]


def _padding_mask(seq_length):
    return pl.pallas_call(
        _mask_body,
        in_specs=[pl.BlockSpec((_B, 1), lambda: (0, 0))],
        out_specs=pl.BlockSpec((_B, _NUM_SAMPLE), lambda: (0, 0)),
        out_shape=jax.ShapeDtypeStruct((_B, _NUM_SAMPLE), jnp.bool_),
    )(seq_length.astype(jnp.int32).reshape(_B, 1))


def _gather_body(
    nodes_hbm,
    unique_hbm,
    samp_hbm,
    proj_hbm,
    nodes_out,
    neighs_out,
    sidx_v,
    cidx_v,
    nidx_v,
    rows_v,
    sem,
):
    wid = lax.axis_index("s") * _NC + lax.axis_index("c")
    base = wid * _RPW

    pltpu.sync_copy(samp_hbm.at[pl.ds(base, _RPW)], sidx_v)

    pltpu.async_copy(unique_hbm.at[sidx_v], cidx_v, sem).wait()

    def chunk(c, carry):
        off = c * _CH
        pltpu.async_copy(
            proj_hbm.at[cidx_v.at[pl.ds(off, _CH)]], rows_v, sem
        ).wait()
        pltpu.sync_copy(rows_v, neighs_out.at[pl.ds(base + off, _CH)])
        return carry

    lax.fori_loop(0, _NCH, chunk, 0)

    nbase = wid * _NPW
    pltpu.sync_copy(nodes_hbm.at[pl.ds(nbase, _NPW)], nidx_v)
    pltpu.async_copy(
        proj_hbm.at[nidx_v], rows_v.at[pl.ds(0, _NPW)], sem
    ).wait()
    pltpu.sync_copy(rows_v.at[pl.ds(0, _NPW)], nodes_out.at[pl.ds(nbase, _NPW)])


def _sc_gather(nodes, unique_nodes_list, samp_flat, proj):
    mesh = plsc.VectorSubcoreMesh(core_axis_name="c", subcore_axis_name="s")
    f = functools.partial(
        pl.kernel,
        mesh=mesh,
        out_type=[
            jax.ShapeDtypeStruct((_B, _EMB), jnp.float32),
            jax.ShapeDtypeStruct((_TOTAL_NEIGHS, _EMB), jnp.float32),
        ],
        scratch_types=[
            pltpu.VMEM((_RPW,), jnp.int32),
            pltpu.VMEM((_RPW,), jnp.int32),
            pltpu.VMEM((_NPW,), jnp.int32),
            pltpu.VMEM((_CH, _EMB), jnp.float32),
            pltpu.SemaphoreType.DMA,
        ],
    )(_gather_body)
    return f(nodes, unique_nodes_list, samp_flat, proj)


def kernel(nodes, unique_nodes_list, samp_neighs_t, seq_length, features, W_in, b_in):
    proj = _project_features(features, W_in, b_in)

    samp_flat = samp_neighs_t.reshape(-1).astype(jnp.int32)
    nodes_emb, neighs_flat = _sc_gather(
        nodes.astype(jnp.int32),
        unique_nodes_list.astype(jnp.int32),
        samp_flat,
        proj,
    )
    neighs_emb = neighs_flat.reshape(_B, _NUM_SAMPLE, _EMB)
    padding_mask = _padding_mask(seq_length)
    return (nodes_emb, neighs_emb, samp_neighs_t, padding_mask)
